# baseline (speedup 1.0000x reference)
import functools

import jax
import jax.numpy as jnp
from jax import lax
from jax.experimental import pallas as pl
from jax.experimental.pallas import tpu as pltpu
from jax.experimental.pallas import tpu_sc as plsc

BATCH = 16384
DIM = 128
LANES = 16
NC = 2
NS = 16
NW = NC * NS
CHUNK = 128
B_PER_W = BATCH // NW
NCHUNK = B_PER_W // CHUNK


def _sc_body(idx_t_hbm, idx_c_hbm, ww_hbm, wc_hbm, aff_hbm, out_hbm,
             idx_t_v, idx_c_v, wbuf, cbuf, out_v, aff_v,
             sem_w, sem_c, sem_o):
    wid = lax.axis_index("s") * NC + lax.axis_index("c")

    pltpu.sync_copy(idx_t_hbm.at[wid], idx_t_v)
    pltpu.sync_copy(idx_c_hbm.at[wid], idx_c_v)
    pltpu.sync_copy(aff_hbm, aff_v)

    dw = aff_v[0, :]
    db = aff_v[1, :]
    lane = lax.iota(jnp.int32, LANES)

    for j in range(NCHUNK):
        cp_w = pltpu.async_copy(ww_hbm.at[idx_t_v.at[j]], wbuf, sem_w)
        cp_c = pltpu.async_copy(wc_hbm.at[idx_c_v.at[j]], cbuf, sem_c)
        cp_w.wait()
        cp_c.wait()

        def group_body(g, _, j=j):
            rows = g * LANES + lane

            def col_body(k, acc):
                col = jnp.zeros((LANES,), jnp.int32) + k
                wv = plsc.load_gather(wbuf, [rows, col])
                cv = plsc.load_gather(cbuf, [rows, col])
                return acc + wv * cv

            acc = lax.fori_loop(0, DIM, col_body,
                                jnp.zeros((LANES,), jnp.float32), unroll=8)
            z = acc * dw + db
            out_v[pl.ds(j * CHUNK + g * LANES, LANES)] = (
                1.0 / (1.0 + jnp.exp(-z)))
            return _

        lax.fori_loop(0, CHUNK // LANES, group_body, None)

    pltpu.async_copy(out_v, out_hbm.at[pl.ds(wid * B_PER_W, B_PER_W)],
                     sem_o).wait()


@jax.jit
def _sc_call(idx_t, idx_c, W_word, W_ctx, affine):
    mesh = plsc.VectorSubcoreMesh(core_axis_name="c", subcore_axis_name="s")
    f = functools.partial(
        pl.kernel,
        mesh=mesh,
        out_type=jax.ShapeDtypeStruct((BATCH,), jnp.float32),
        compiler_params=pltpu.CompilerParams(needs_layout_passes=False),
        scratch_types=[
            pltpu.VMEM((NCHUNK, CHUNK), jnp.int32),
            pltpu.VMEM((NCHUNK, CHUNK), jnp.int32),
            pltpu.VMEM((CHUNK, DIM), jnp.float32),
            pltpu.VMEM((CHUNK, DIM), jnp.float32),
            pltpu.VMEM((B_PER_W,), jnp.float32),
            pltpu.VMEM((2, LANES), jnp.float32),
            pltpu.SemaphoreType.DMA,
            pltpu.SemaphoreType.DMA,
            pltpu.SemaphoreType.DMA,
        ],
    )(_sc_body)
    return f(idx_t, idx_c, W_word, W_ctx, affine)


def kernel(word_target, word_context, W_word, W_ctx, dense_w, dense_b):
    idx_t = word_target.reshape(NW, NCHUNK, CHUNK)
    idx_c = word_context.reshape(NW, NCHUNK, CHUNK)
    affine = jnp.stack([
        jnp.full((LANES,), dense_w[0, 0], dtype=jnp.float32),
        jnp.full((LANES,), dense_b[0], dtype=jnp.float32),
    ])
    out = _sc_call(idx_t, idx_c, W_word, W_ctx, affine)
    return out.reshape(BATCH, 1)


# speedup vs baseline: 2.5743x; 2.5743x over previous
import functools

import jax
import jax.numpy as jnp
from jax import lax
from jax.experimental import pallas as pl
from jax.experimental.pallas import tpu as pltpu
from jax.experimental.pallas import tpu_sc as plsc

BATCH = 16384
DIM = 128
LANES = 16
NC = 2
NS = 16
NW = NC * NS
CHUNK = 128
B_PER_W = BATCH // NW
NCHUNK = B_PER_W // CHUNK


def _sc_body(idx_t_hbm, idx_c_hbm, ww_hbm, wc_hbm, aff_hbm, out_hbm,
             idx_t_v, idx_c_v, wbuf, cbuf, out_v, aff_v,
             sem_w, sem_c, sem_o):
    wid = lax.axis_index("s") * NC + lax.axis_index("c")

    pltpu.sync_copy(idx_t_hbm.at[wid], idx_t_v)
    pltpu.sync_copy(idx_c_hbm.at[wid], idx_c_v)
    pltpu.sync_copy(aff_hbm, aff_v)

    dw = aff_v[0, :]
    db = aff_v[1, :]
    lane = lax.iota(jnp.int32, LANES)

    for j in range(NCHUNK):
        cp_w = pltpu.async_copy(ww_hbm.at[idx_t_v.at[j]], wbuf, sem_w)
        cp_c = pltpu.async_copy(wc_hbm.at[idx_c_v.at[j]], cbuf, sem_c)
        cp_w.wait()
        cp_c.wait()

        def group_body(g, _, j=j):
            rows = g * LANES + lane

            def col_body(k, acc):
                col = (lane + k) & (DIM - 1)
                wv = plsc.load_gather(wbuf, [rows, col])
                cv = plsc.load_gather(cbuf, [rows, col])
                return acc + wv * cv

            acc = lax.fori_loop(0, DIM, col_body,
                                jnp.zeros((LANES,), jnp.float32), unroll=8)
            z = acc * dw + db
            out_v[pl.ds(j * CHUNK + g * LANES, LANES)] = (
                1.0 / (1.0 + jnp.exp(-z)))
            return _

        lax.fori_loop(0, CHUNK // LANES, group_body, None)

    pltpu.async_copy(out_v, out_hbm.at[pl.ds(wid * B_PER_W, B_PER_W)],
                     sem_o).wait()


@jax.jit
def _sc_call(idx_t, idx_c, W_word, W_ctx, affine):
    mesh = plsc.VectorSubcoreMesh(core_axis_name="c", subcore_axis_name="s")
    f = functools.partial(
        pl.kernel,
        mesh=mesh,
        out_type=jax.ShapeDtypeStruct((BATCH,), jnp.float32),
        compiler_params=pltpu.CompilerParams(needs_layout_passes=False),
        scratch_types=[
            pltpu.VMEM((NCHUNK, CHUNK), jnp.int32),
            pltpu.VMEM((NCHUNK, CHUNK), jnp.int32),
            pltpu.VMEM((CHUNK, DIM), jnp.float32),
            pltpu.VMEM((CHUNK, DIM), jnp.float32),
            pltpu.VMEM((B_PER_W,), jnp.float32),
            pltpu.VMEM((2, LANES), jnp.float32),
            pltpu.SemaphoreType.DMA,
            pltpu.SemaphoreType.DMA,
            pltpu.SemaphoreType.DMA,
        ],
    )(_sc_body)
    return f(idx_t, idx_c, W_word, W_ctx, affine)


def kernel(word_target, word_context, W_word, W_ctx, dense_w, dense_b):
    idx_t = word_target.reshape(NW, NCHUNK, CHUNK)
    idx_c = word_context.reshape(NW, NCHUNK, CHUNK)
    affine = jnp.stack([
        jnp.full((LANES,), dense_w[0, 0], dtype=jnp.float32),
        jnp.full((LANES,), dense_b[0], dtype=jnp.float32),
    ])
    out = _sc_call(idx_t, idx_c, W_word, W_ctx, affine)
    return out.reshape(BATCH, 1)


# speedup vs baseline: 2.8183x; 1.0948x over previous
import functools

import jax
import jax.numpy as jnp
from jax import lax
from jax.experimental import pallas as pl
from jax.experimental.pallas import tpu as pltpu
from jax.experimental.pallas import tpu_sc as plsc

BATCH = 16384
DIM = 128
LANES = 16
NC = 2
NS = 16
NW = NC * NS
CHUNK = 128
B_PER_W = BATCH // NW
NCHUNK = B_PER_W // CHUNK


def _sc_body(idx_t_hbm, idx_c_hbm, ww_hbm, wc_hbm, aff_hbm, out_hbm,
             idx_t_v, idx_c_v, wbuf0, cbuf0, wbuf1, cbuf1, out_v, aff_v,
             sem_w0, sem_c0, sem_w1, sem_c1, sem_o):
    wid = lax.axis_index("s") * NC + lax.axis_index("c")

    pltpu.sync_copy(idx_t_hbm.at[wid], idx_t_v)
    pltpu.sync_copy(idx_c_hbm.at[wid], idx_c_v)
    pltpu.sync_copy(aff_hbm, aff_v)

    dw = aff_v[0, :]
    db = aff_v[1, :]
    lane = lax.iota(jnp.int32, LANES)

    bufs = ((wbuf0, cbuf0, sem_w0, sem_c0), (wbuf1, cbuf1, sem_w1, sem_c1))

    def fire(j):
        wb, cb, sw, sc_ = bufs[j % 2]
        hw = pltpu.async_copy(ww_hbm.at[idx_t_v.at[j]], wb, sw)
        hc = pltpu.async_copy(wc_hbm.at[idx_c_v.at[j]], cb, sc_)
        return hw, hc

    handles = [fire(0)]
    for j in range(NCHUNK):
        if j + 1 < NCHUNK:
            handles.append(fire(j + 1))
        hw, hc = handles[j]
        hw.wait()
        hc.wait()
        wb, cb, _, _ = bufs[j % 2]

        def group_body(g, _, j=j, wb=wb, cb=cb):
            rows = g * LANES + lane

            def col_body(k, acc):
                col = (lane + k) & (DIM - 1)
                wv = plsc.load_gather(wb, [rows, col])
                cv = plsc.load_gather(cb, [rows, col])
                return acc + wv * cv

            acc = lax.fori_loop(0, DIM, col_body,
                                jnp.zeros((LANES,), jnp.float32), unroll=8)
            z = acc * dw + db
            out_v[pl.ds(j * CHUNK + g * LANES, LANES)] = (
                1.0 / (1.0 + jnp.exp(-z)))
            return _

        lax.fori_loop(0, CHUNK // LANES, group_body, None)

    pltpu.async_copy(out_v, out_hbm.at[pl.ds(wid * B_PER_W, B_PER_W)],
                     sem_o).wait()


@jax.jit
def _sc_call(idx_t, idx_c, W_word, W_ctx, affine):
    mesh = plsc.VectorSubcoreMesh(core_axis_name="c", subcore_axis_name="s")
    f = functools.partial(
        pl.kernel,
        mesh=mesh,
        out_type=jax.ShapeDtypeStruct((BATCH,), jnp.float32),
        compiler_params=pltpu.CompilerParams(
            needs_layout_passes=False,
            disable_bounds_checks=True,
            disable_semaphore_checks=True,
            skip_device_barrier=True,
        ),
        scratch_types=[
            pltpu.VMEM((NCHUNK, CHUNK), jnp.int32),
            pltpu.VMEM((NCHUNK, CHUNK), jnp.int32),
            pltpu.VMEM((CHUNK, DIM), jnp.float32),
            pltpu.VMEM((CHUNK, DIM), jnp.float32),
            pltpu.VMEM((CHUNK, DIM), jnp.float32),
            pltpu.VMEM((CHUNK, DIM), jnp.float32),
            pltpu.VMEM((B_PER_W,), jnp.float32),
            pltpu.VMEM((2, LANES), jnp.float32),
            pltpu.SemaphoreType.DMA,
            pltpu.SemaphoreType.DMA,
            pltpu.SemaphoreType.DMA,
            pltpu.SemaphoreType.DMA,
            pltpu.SemaphoreType.DMA,
        ],
    )(_sc_body)
    return f(idx_t, idx_c, W_word, W_ctx, affine)


def kernel(word_target, word_context, W_word, W_ctx, dense_w, dense_b):
    idx_t = word_target.reshape(NW, NCHUNK, CHUNK)
    idx_c = word_context.reshape(NW, NCHUNK, CHUNK)
    affine = jnp.stack([
        jnp.full((LANES,), dense_w[0, 0], dtype=jnp.float32),
        jnp.full((LANES,), dense_b[0], dtype=jnp.float32),
    ])
    out = _sc_call(idx_t, idx_c, W_word, W_ctx, affine)
    return out.reshape(BATCH, 1)


# speedup vs baseline: 2.8850x; 1.0237x over previous
import functools

import jax
import jax.numpy as jnp
from jax import lax
from jax.experimental import pallas as pl
from jax.experimental.pallas import tpu as pltpu
from jax.experimental.pallas import tpu_sc as plsc

BATCH = 16384
DIM = 128
LANES = 16
NC = 2
NS = 16
NW = NC * NS
CHUNK = 128
B_PER_W = BATCH // NW
NCHUNK = B_PER_W // CHUNK


def _sc_body(idx_t_hbm, idx_c_hbm, ww_hbm, wc_hbm, aff_hbm, out_hbm,
             idx_t_v, idx_c_v, wbuf0, cbuf0, wbuf1, cbuf1, out_v, aff_v,
             sem_w0, sem_c0, sem_w1, sem_c1, sem_o):
    wid = lax.axis_index("s") * NC + lax.axis_index("c")

    h_it = pltpu.async_copy(idx_t_hbm.at[wid], idx_t_v, sem_o)
    h_ic = pltpu.async_copy(idx_c_hbm.at[wid], idx_c_v, sem_o)
    h_af = pltpu.async_copy(aff_hbm, aff_v, sem_o)
    h_it.wait()
    h_ic.wait()
    h_af.wait()

    dw = aff_v[0, :]
    db = aff_v[1, :]
    lane = lax.iota(jnp.int32, LANES)

    bufs = ((wbuf0, cbuf0, sem_w0, sem_c0), (wbuf1, cbuf1, sem_w1, sem_c1))

    def fire(j):
        wb, cb, sw, sc_ = bufs[j % 2]
        hw = pltpu.async_copy(ww_hbm.at[idx_t_v.at[j]], wb, sw)
        hc = pltpu.async_copy(wc_hbm.at[idx_c_v.at[j]], cb, sc_)
        return hw, hc

    handles = [fire(0)]
    for j in range(NCHUNK):
        if j + 1 < NCHUNK:
            handles.append(fire(j + 1))
        hw, hc = handles[j]
        hw.wait()
        hc.wait()
        wb, cb, _, _ = bufs[j % 2]

        def group_body(g, _, j=j, wb=wb, cb=cb):
            rows = g * LANES + lane

            def col_body(k, acc):
                col = (lane + k) & (DIM - 1)
                wv = plsc.load_gather(wb, [rows, col])
                cv = plsc.load_gather(cb, [rows, col])
                return acc + wv * cv

            acc = lax.fori_loop(0, DIM, col_body,
                                jnp.zeros((LANES,), jnp.float32), unroll=16)
            z = acc * dw + db
            out_v[pl.ds(j * CHUNK + g * LANES, LANES)] = (
                1.0 / (1.0 + jnp.exp(-z)))
            return _

        lax.fori_loop(0, CHUNK // LANES, group_body, None)

    pltpu.async_copy(out_v, out_hbm.at[pl.ds(wid * B_PER_W, B_PER_W)],
                     sem_o).wait()


@jax.jit
def _sc_call(idx_t, idx_c, W_word, W_ctx, affine):
    mesh = plsc.VectorSubcoreMesh(core_axis_name="c", subcore_axis_name="s")
    f = functools.partial(
        pl.kernel,
        mesh=mesh,
        out_type=jax.ShapeDtypeStruct((BATCH,), jnp.float32),
        compiler_params=pltpu.CompilerParams(
            needs_layout_passes=False,
            disable_bounds_checks=True,
            disable_semaphore_checks=True,
            skip_device_barrier=True,
        ),
        scratch_types=[
            pltpu.VMEM((NCHUNK, CHUNK), jnp.int32),
            pltpu.VMEM((NCHUNK, CHUNK), jnp.int32),
            pltpu.VMEM((CHUNK, DIM), jnp.float32),
            pltpu.VMEM((CHUNK, DIM), jnp.float32),
            pltpu.VMEM((CHUNK, DIM), jnp.float32),
            pltpu.VMEM((CHUNK, DIM), jnp.float32),
            pltpu.VMEM((B_PER_W,), jnp.float32),
            pltpu.VMEM((2, LANES), jnp.float32),
            pltpu.SemaphoreType.DMA,
            pltpu.SemaphoreType.DMA,
            pltpu.SemaphoreType.DMA,
            pltpu.SemaphoreType.DMA,
            pltpu.SemaphoreType.DMA,
        ],
    )(_sc_body)
    return f(idx_t, idx_c, W_word, W_ctx, affine)


def kernel(word_target, word_context, W_word, W_ctx, dense_w, dense_b):
    idx_t = word_target.reshape(NW, NCHUNK, CHUNK)
    idx_c = word_context.reshape(NW, NCHUNK, CHUNK)
    affine = jnp.stack([
        jnp.full((LANES,), dense_w[0, 0], dtype=jnp.float32),
        jnp.full((LANES,), dense_b[0], dtype=jnp.float32),
    ])
    out = _sc_call(idx_t, idx_c, W_word, W_ctx, affine)
    return out.reshape(BATCH, 1)
